# baseline (speedup 1.0000x reference)
import functools

import jax
import jax.numpy as jnp
from jax.experimental import pallas as pl
from jax.experimental.pallas import tpu as pltpu

N, I, O = 10000, 128, 128
BM = 240


def _body(ai0_ref, ai1_ref, ao0_ref, ao1_ref, x_ref, w_ref, b_ref, c_ref,
          out_ref, p_ref, q_ref, bias_ref):
    @pl.when(pl.program_id(0) == 0)
    def _prologue():
        c_in = c_ref[0, 0]
        c_out = c_ref[0, 1]
        w_all = w_ref[0]
        wi = (w_ref[1] + w_all) * c_in
        wo = (w_ref[2] + w_all) * c_out
        x = x_ref[...]
        p_ref[...] = jnp.dot(x, wi, preferred_element_type=jnp.float32)
        q_ref[...] = jnp.dot(x, wo, preferred_element_type=jnp.float32)
        b_all = b_ref[0, :]
        b_in = b_ref[1, :]
        b_out = b_ref[2, :]
        bias_ref[...] = (c_in * (b_in + b_all)
                         + c_out * (b_out + b_all))[None, :]

    p = p_ref[...]
    q = q_ref[...]
    h = BM // 2
    acc0 = jnp.dot(ai0_ref[...], p, preferred_element_type=jnp.float32)
    acc0 += jnp.dot(ao0_ref[...], q, preferred_element_type=jnp.float32)
    out_ref[:h, :] = acc0 + bias_ref[...]
    acc1 = jnp.dot(ai1_ref[...], p, preferred_element_type=jnp.float32)
    acc1 += jnp.dot(ao1_ref[...], q, preferred_element_type=jnp.float32)
    out_ref[h:, :] = acc1 + bias_ref[...]


@functools.partial(jax.jit, static_argnames=())
def kernel(x, ai, ao, W_all, b_all, W_in, b_in, W_out, b_out, C_in, C_out):
    c = jnp.stack([C_in[0], C_out[0]])[None, :]
    b = jnp.stack([b_all, b_in, b_out])
    w = jnp.stack([W_all, W_in, W_out])

    grid = (pl.cdiv(N, BM),)
    out = pl.pallas_call(
        _body,
        grid=grid,
        in_specs=[
            pl.BlockSpec((BM // 2, N), lambda i: (2 * i, 0)),
            pl.BlockSpec((BM // 2, N), lambda i: (2 * i + 1, 0)),
            pl.BlockSpec((BM // 2, N), lambda i: (2 * i, 0)),
            pl.BlockSpec((BM // 2, N), lambda i: (2 * i + 1, 0)),
            pl.BlockSpec((N, I), lambda i: (0, 0)),
            pl.BlockSpec((3, I, O), lambda i: (0, 0, 0)),
            pl.BlockSpec((3, O), lambda i: (0, 0)),
            pl.BlockSpec((1, 2), lambda i: (0, 0)),
        ],
        out_specs=pl.BlockSpec((BM, O), lambda i: (i, 0)),
        out_shape=jax.ShapeDtypeStruct((N, O), jnp.float32),
        scratch_shapes=[
            pltpu.VMEM((N, O), jnp.float32),
            pltpu.VMEM((N, O), jnp.float32),
            pltpu.VMEM((1, O), jnp.float32),
        ],
    )(ai, ai, ao, ao, x, w, b, c)
    return out


# speedup vs baseline: 1.0088x; 1.0088x over previous
import functools

import jax
import jax.numpy as jnp
from jax.experimental import pallas as pl
from jax.experimental.pallas import tpu as pltpu

N, I, O = 10000, 128, 128
BM = 200


def _body(ai_ref, ao_ref, x_ref, w_ref, b_ref, c_ref,
          out_ref, p_ref, q_ref, bias_ref):
    @pl.when(pl.program_id(0) == 0)
    def _prologue():
        c_in = c_ref[0, 0]
        c_out = c_ref[0, 1]
        w_all = w_ref[0]
        wi = (w_ref[1] + w_all) * c_in
        wo = (w_ref[2] + w_all) * c_out
        x = x_ref[...]
        p_ref[...] = jnp.dot(x, wi, preferred_element_type=jnp.float32)
        q_ref[...] = jnp.dot(x, wo, preferred_element_type=jnp.float32)
        b_all = b_ref[0, :]
        b_in = b_ref[1, :]
        b_out = b_ref[2, :]
        bias_ref[...] = (c_in * (b_in + b_all)
                         + c_out * (b_out + b_all))[None, :]

    acc = jnp.dot(ai_ref[...], p_ref[...], preferred_element_type=jnp.float32)
    acc += jnp.dot(ao_ref[...], q_ref[...], preferred_element_type=jnp.float32)
    out_ref[...] = acc + bias_ref[...]


@functools.partial(jax.jit, static_argnames=())
def kernel(x, ai, ao, W_all, b_all, W_in, b_in, W_out, b_out, C_in, C_out):
    c = jnp.stack([C_in[0], C_out[0]])[None, :]
    b = jnp.stack([b_all, b_in, b_out])
    w = jnp.stack([W_all, W_in, W_out])

    grid = (pl.cdiv(N, BM),)
    out = pl.pallas_call(
        _body,
        grid=grid,
        in_specs=[
            pl.BlockSpec((BM, N), lambda i: (i, 0)),
            pl.BlockSpec((BM, N), lambda i: (i, 0)),
            pl.BlockSpec((N, I), lambda i: (0, 0)),
            pl.BlockSpec((3, I, O), lambda i: (0, 0, 0)),
            pl.BlockSpec((3, O), lambda i: (0, 0)),
            pl.BlockSpec((1, 2), lambda i: (0, 0)),
        ],
        out_specs=pl.BlockSpec((BM, O), lambda i: (i, 0)),
        out_shape=jax.ShapeDtypeStruct((N, O), jnp.float32),
        scratch_shapes=[
            pltpu.VMEM((N, O), jnp.float32),
            pltpu.VMEM((N, O), jnp.float32),
            pltpu.VMEM((1, O), jnp.float32),
        ],
    )(ai, ao, x, w, b, c)
    return out
